# baseline (speedup 1.0000x reference)
import functools

import jax
import jax.numpy as jnp
from jax import lax
from jax.experimental import pallas as pl
from jax.experimental.pallas import tpu as pltpu
from jax.experimental.pallas import tpu_sc as plsc

_L = 16
_NC = 2
_NS = 16
_NW = _NC * _NS


def _neg_sqrt(x):
    xc = jnp.maximum(x, jnp.float32(1e-30))
    i = plsc.bitcast(xc, jnp.int32)
    i = jnp.int32(0x5F3759DF) - lax.shift_right_logical(i, 1)
    y = plsc.bitcast(i, jnp.float32)
    half = jnp.float32(0.5) * xc
    for _ in range(3):
        y = y * (jnp.float32(1.5) - half * y * y)
    return -(x * y)


def _tec_kernel(heads_hbm, rels_hbm, tails_hbm, etab_hbm, rtab_hbm, out_hbm,
                hidx, ridx, tidx, hrows, rrows, trows, outv, sem):
    bpw = hidx.shape[0]
    d = etab_hbm.shape[1]
    wid = lax.axis_index("s") * _NC + lax.axis_index("c")
    base = wid * bpw

    pltpu.sync_copy(heads_hbm.at[pl.ds(base, bpw)], hidx)
    pltpu.sync_copy(rels_hbm.at[pl.ds(base, bpw)], ridx)
    pltpu.sync_copy(tails_hbm.at[pl.ds(base, bpw)], tidx)

    cps = [
        pltpu.async_copy(etab_hbm.at[hidx], hrows, sem),
        pltpu.async_copy(rtab_hbm.at[ridx], rrows, sem),
        pltpu.async_copy(etab_hbm.at[tidx], trows, sem),
    ]
    for cp in cps:
        cp.wait()

    def group_body(g, carry):
        rows16 = g * _L + lax.iota(jnp.int32, _L)
        acc = jnp.zeros((_L,), jnp.float32)
        for j in range(d):
            colj = jnp.full((_L,), j, jnp.int32)
            hv = plsc.load_gather(hrows, [rows16, colj])
            rv = plsc.load_gather(rrows, [rows16, colj])
            tv = plsc.load_gather(trows, [rows16, colj])
            dlt = hv + rv - tv
            acc = acc + dlt * dlt
        outv[pl.ds(g * _L, _L)] = _neg_sqrt(acc)
        return carry

    lax.fori_loop(0, bpw // _L, group_body, 0)
    pltpu.sync_copy(outv, out_hbm.at[pl.ds(base, bpw)])


def kernel(heads, relations, tails, entity_embeddings, relation_embeddings):
    batch = heads.shape[0]
    dim = entity_embeddings.shape[1]
    assert batch % (8 * _NW) == 0
    bpw = batch // _NW

    mesh = plsc.VectorSubcoreMesh(core_axis_name="c", subcore_axis_name="s")
    kern = functools.partial(
        pl.kernel,
        mesh=mesh,
        out_type=jax.ShapeDtypeStruct((batch,), jnp.float32),
        scratch_types=[
            pltpu.VMEM((bpw,), jnp.int32),
            pltpu.VMEM((bpw,), jnp.int32),
            pltpu.VMEM((bpw,), jnp.int32),
            pltpu.VMEM((bpw, dim), jnp.float32),
            pltpu.VMEM((bpw, dim), jnp.float32),
            pltpu.VMEM((bpw, dim), jnp.float32),
            pltpu.VMEM((bpw,), jnp.float32),
            pltpu.SemaphoreType.DMA,
        ],
        compiler_params=pltpu.CompilerParams(
            needs_layout_passes=False, use_tc_tiling_on_sc=False),
    )(_tec_kernel)
    return kern(heads.astype(jnp.int32), relations.astype(jnp.int32),
                tails.astype(jnp.int32), entity_embeddings,
                relation_embeddings)
